# baseline (speedup 1.0000x reference)
import jax
import jax.numpy as jnp
from jax import lax
from jax.experimental import pallas as pl
from jax.experimental.pallas import tpu as pltpu
from jax.experimental.pallas import tpu_sc as plsc
from jax._src.pallas import mpmd as _mpmd

NX, NY, C, B, P = 512, 512, 64, 4, 40000
R = B * NX * NY
CW = 128
NW = 32
PPAD = 40960
NPW = PPAD // NW
SCH = 128
NCHUNK = NPW // SCH
LANES = 16

XG = 4


def _flag_zero_body(o_ref):
    o_ref[...] = jnp.zeros_like(o_ref)


def _zero_flags():
    return pl.pallas_call(
        _flag_zero_body,
        out_shape=jax.ShapeDtypeStruct((R,), jnp.int32),
        grid=(16,),
        out_specs=pl.BlockSpec((R // 16,), lambda i: (i,)),
    )()


def _scatter_body(feat_hbm, b_hbm, y_hbm, x_hbm, flags0_hbm,
                  canvas_hbm, flags_hbm,
                  bv, yv, xv, rows2d, ones2d, fbuf, sem, fsem):
    del flags0_hbm
    wid = lax.axis_index("s") * 2 + lax.axis_index("c")
    base = wid * NPW
    pltpu.sync_copy(b_hbm.at[pl.ds(base, NPW)], bv)
    pltpu.sync_copy(y_hbm.at[pl.ds(base, NPW)], yv)
    pltpu.sync_copy(x_hbm.at[pl.ds(base, NPW)], xv)

    def compute_rows(i, _):
        off = i * LANES
        vb = bv[pl.ds(off, LANES)]
        vy = yv[pl.ds(off, LANES)]
        vx = xv[pl.ds(off, LANES)]
        row = vb * (NX * NY) + vx * NY + vy
        r = i // (SCH // LANES)
        c = (i % (SCH // LANES)) * LANES
        rows2d[r, pl.ds(c, LANES)] = row
        ones2d[r, pl.ds(c, LANES)] = jnp.ones((LANES,), jnp.int32)
        return _

    lax.fori_loop(0, NPW // LANES, compute_rows, None)

    hs = []
    fhs = []
    for j in range(NCHUNK):
        if j >= 2:
            hs[j - 2].wait()
        pltpu.sync_copy(feat_hbm.at[pl.ds(base + j * SCH, SCH)],
                        fbuf.at[j % 2, :, pl.ds(0, C)])
        h = pltpu.make_async_copy(
            fbuf.at[j % 2],
            canvas_hbm.at[rows2d.at[j]],
            sem,
        )
        h.start()
        hs.append(h)
        fh = pltpu.make_async_copy(
            ones2d.at[j],
            flags_hbm.at[rows2d.at[j]],
            fsem,
        )
        fh.start()
        fhs.append(fh)
    hs[-2].wait()
    hs[-1].wait()
    for fh in fhs:
        fh.wait()


def _scatter_sc(feat, b_arr, y_arr, x_arr, flags0):
    mesh = plsc.VectorSubcoreMesh(core_axis_name="c", subcore_axis_name="s")
    fn = _mpmd._mpmd_map(
        [(mesh, _scatter_body)],
        (jax.ShapeDtypeStruct((R, CW), jnp.bfloat16),
         jax.ShapeDtypeStruct((R,), jnp.int32)),
        input_output_aliases={4: 1},
        scratch_types=[
            pltpu.VMEM((NPW,), jnp.int32),
            pltpu.VMEM((NPW,), jnp.int32),
            pltpu.VMEM((NPW,), jnp.int32),
            pltpu.VMEM((NCHUNK, SCH), jnp.int32),
            pltpu.VMEM((NCHUNK, SCH), jnp.int32),
            pltpu.VMEM((2, SCH, CW), jnp.bfloat16),
            pltpu.SemaphoreType.DMA,
            pltpu.SemaphoreType.DMA,
        ],
        compiler_params=pltpu.CompilerParams(use_tc_tiling_on_sc=False),
        interpret=False,
        debug=False,
        cost_estimate=None,
        name="pillar_scatter_sc",
        metadata=None,
    )
    return fn(feat, b_arr, y_arr, x_arr, flags0)


def _retile_body(cv_ref, fl_ref, o_ref):
    f = fl_ref[0]
    xf = cv_ref[...].reshape(XG * NY, CW)
    for k in range(XG):
        sub = xf[k * NY:(k + 1) * NY, :C]
        xt = sub.astype(jnp.float32).T
        mk = (f[k] != 0)[None, :]
        o_ref[0, k] = jnp.where(mk, xt, jnp.zeros_like(xt))


def _retile(canvas3, flags2d):
    grid = B * NX // XG
    return pl.pallas_call(
        _retile_body,
        out_shape=jax.ShapeDtypeStruct((B, NX, C, NY), jnp.float32),
        grid=(grid,),
        in_specs=[
            pl.BlockSpec((XG * NY // 16, 16, 128), lambda i: (i, 0, 0)),
            pl.BlockSpec((1, XG, NY), lambda i: (i, 0, 0)),
        ],
        out_specs=pl.BlockSpec((1, XG, C, NY), lambda i: (i // (NX // XG), i % (NX // XG), 0, 0)),
    )(canvas3, flags2d)


def kernel(pillar_features, coors):
    coors = coors.astype(jnp.int32)
    pad = PPAD - P
    feat = jnp.concatenate(
        [pillar_features,
         jnp.broadcast_to(pillar_features[0], (pad, C))], axis=0)
    feat = feat.astype(jnp.bfloat16)
    cpad = jnp.concatenate(
        [coors, jnp.broadcast_to(coors[0], (pad, 3))], axis=0)
    b_arr = cpad[:, 0]
    y_arr = cpad[:, 1]
    x_arr = cpad[:, 2]
    flags0 = _zero_flags()
    canvas, flags = _scatter_sc(feat, b_arr, y_arr, x_arr, flags0)
    canvas3 = canvas.reshape(R * CW // (16 * 128), 16, 128)
    out_t = _retile(canvas3, flags.reshape(B * NX // XG, XG, NY))
    return jnp.transpose(out_t, (0, 1, 3, 2))


# speedup vs baseline: 2.2096x; 2.2096x over previous
import jax
import jax.numpy as jnp
from jax import lax
from jax.experimental import pallas as pl
from jax.experimental.pallas import tpu as pltpu
from jax.experimental.pallas import tpu_sc as plsc
from jax._src.pallas import mpmd as _mpmd

NX, NY, C, B, P = 512, 512, 64, 4, 40000
R = B * NX * NY
CW = 128
NW = 32
PPAD = 40960
NPW = PPAD // NW
SCH = 128
NCHUNK = NPW // SCH
LANES = 16

XG = 4


def _flag_zero_body(o_ref):
    o_ref[...] = jnp.zeros_like(o_ref)


def _zero_flags():
    return pl.pallas_call(
        _flag_zero_body,
        out_shape=jax.ShapeDtypeStruct((R,), jnp.int32),
        grid=(16,),
        out_specs=pl.BlockSpec((R // 16,), lambda i: (i,)),
    )()


def _scatter_body(feat_hbm, b_hbm, y_hbm, x_hbm, flags0_hbm,
                  canvas_hbm, flags_hbm,
                  bv, yv, xv, rows2d, ones2d, fbuf, sem, fsem):
    del flags0_hbm
    wid = lax.axis_index("s") * 2 + lax.axis_index("c")
    base = wid * NPW
    pltpu.sync_copy(b_hbm.at[pl.ds(base, NPW)], bv)
    pltpu.sync_copy(y_hbm.at[pl.ds(base, NPW)], yv)
    pltpu.sync_copy(x_hbm.at[pl.ds(base, NPW)], xv)

    def compute_rows(i, _):
        off = i * LANES
        vb = bv[pl.ds(off, LANES)]
        vy = yv[pl.ds(off, LANES)]
        vx = xv[pl.ds(off, LANES)]
        row = vb * (NX * NY) + vx * NY + vy
        r = i // (SCH // LANES)
        c = (i % (SCH // LANES)) * LANES
        rows2d[r, pl.ds(c, LANES)] = row
        ones2d[r, pl.ds(c, LANES)] = jnp.ones((LANES,), jnp.int32)
        return _

    lax.fori_loop(0, NPW // LANES, compute_rows, None)

    fhs = []
    for j in range(NCHUNK):
        fh = pltpu.make_async_copy(
            ones2d.at[j],
            flags_hbm.at[rows2d.at[j]],
            fsem,
        )
        fh.start()
        fhs.append(fh)
    hs = []
    for j in range(NCHUNK):
        if j >= 2:
            hs[j - 2].wait()
        pltpu.sync_copy(feat_hbm.at[pl.ds(base + j * SCH, SCH)],
                        fbuf.at[j % 2, :, pl.ds(0, C)])
        h = pltpu.make_async_copy(
            fbuf.at[j % 2],
            canvas_hbm.at[rows2d.at[j]],
            sem,
        )
        h.start()
        hs.append(h)
    hs[-2].wait()
    hs[-1].wait()
    for fh in fhs:
        fh.wait()


def _scatter_sc(feat, b_arr, y_arr, x_arr, flags0):
    mesh = plsc.VectorSubcoreMesh(core_axis_name="c", subcore_axis_name="s")
    fn = _mpmd._mpmd_map(
        [(mesh, _scatter_body)],
        (jax.ShapeDtypeStruct((R, CW), jnp.float32),
         jax.ShapeDtypeStruct((R,), jnp.int32)),
        input_output_aliases={4: 1},
        scratch_types=[
            pltpu.VMEM((NPW,), jnp.int32),
            pltpu.VMEM((NPW,), jnp.int32),
            pltpu.VMEM((NPW,), jnp.int32),
            pltpu.VMEM((NCHUNK, SCH), jnp.int32),
            pltpu.VMEM((NCHUNK, SCH), jnp.int32),
            pltpu.VMEM((2, SCH, CW), jnp.float32),
            pltpu.SemaphoreType.DMA,
            pltpu.SemaphoreType.DMA,
        ],
        compiler_params=pltpu.CompilerParams(use_tc_tiling_on_sc=False),
        interpret=False,
        debug=False,
        cost_estimate=None,
        name="pillar_scatter_sc",
        metadata=None,
    )
    return fn(feat, b_arr, y_arr, x_arr, flags0)


def _retile_body(cv_ref, fl_ref, o_ref):
    f = fl_ref[0]
    xf = cv_ref[...].reshape(XG * NY, CW)
    for k in range(XG):
        sub = xf[k * NY:(k + 1) * NY, :C]
        xt = sub.T
        mk = (f[k] != 0)[None, :]
        o_ref[0, k] = jnp.where(mk, xt, jnp.zeros_like(xt))


def _retile(canvas3, flags2d):
    grid = B * NX // XG
    return pl.pallas_call(
        _retile_body,
        out_shape=jax.ShapeDtypeStruct((B, NX, C, NY), jnp.float32),
        grid=(grid,),
        in_specs=[
            pl.BlockSpec((XG * NY // 8, 8, 128), lambda i: (i, 0, 0)),
            pl.BlockSpec((1, XG, NY), lambda i: (i, 0, 0)),
        ],
        out_specs=pl.BlockSpec((1, XG, C, NY), lambda i: (i // (NX // XG), i % (NX // XG), 0, 0)),
    )(canvas3, flags2d)


def kernel(pillar_features, coors):
    coors = coors.astype(jnp.int32)
    pad = PPAD - P
    feat = jnp.concatenate(
        [pillar_features,
         jnp.broadcast_to(pillar_features[0], (pad, C))], axis=0)
    cpad = jnp.concatenate(
        [coors, jnp.broadcast_to(coors[0], (pad, 3))], axis=0)
    b_arr = cpad[:, 0]
    y_arr = cpad[:, 1]
    x_arr = cpad[:, 2]
    flags0 = _zero_flags()
    canvas, flags = _scatter_sc(feat, b_arr, y_arr, x_arr, flags0)
    canvas3 = canvas.reshape(R * CW // (8 * 128), 8, 128)
    out_t = _retile(canvas3, flags.reshape(B * NX // XG, XG, NY))
    return jnp.transpose(out_t, (0, 1, 3, 2))


# speedup vs baseline: 2.2777x; 1.0308x over previous
import jax
import jax.numpy as jnp
from jax import lax
from jax.experimental import pallas as pl
from jax.experimental.pallas import tpu as pltpu
from jax.experimental.pallas import tpu_sc as plsc
from jax._src.pallas import mpmd as _mpmd

NX, NY, C, B, P = 512, 512, 64, 4, 40000
R = B * NX * NY
CW = 128
NW = 32
PPAD = 40960
NPW = PPAD // NW
SCH = 128
NCHUNK = NPW // SCH
LANES = 16

XG = 4


def _flag_zero_body(o_ref):
    o_ref[...] = jnp.zeros_like(o_ref)


def _zero_flags():
    return pl.pallas_call(
        _flag_zero_body,
        out_shape=jax.ShapeDtypeStruct((R,), jnp.int32),
        grid=(16,),
        out_specs=pl.BlockSpec((R // 16,), lambda i: (i,)),
    )()


def _scatter_body(feat_hbm, b_hbm, y_hbm, x_hbm, flags0_hbm,
                  canvas_hbm, flags_hbm,
                  bv, yv, xv, rows2d, ones2d, fbuf, sem, fsem):
    del flags0_hbm
    wid = lax.axis_index("s") * 2 + lax.axis_index("c")
    base = wid * NPW
    pltpu.sync_copy(b_hbm.at[pl.ds(base, NPW)], bv)
    pltpu.sync_copy(y_hbm.at[pl.ds(base, NPW)], yv)
    pltpu.sync_copy(x_hbm.at[pl.ds(base, NPW)], xv)

    def compute_rows(i, _):
        off = i * LANES
        vb = bv[pl.ds(off, LANES)]
        vy = yv[pl.ds(off, LANES)]
        vx = xv[pl.ds(off, LANES)]
        row = vb * (NX * NY) + vx * NY + vy
        r = i // (SCH // LANES)
        c = (i % (SCH // LANES)) * LANES
        rows2d[r, pl.ds(c, LANES)] = row
        ones2d[r, pl.ds(c, LANES)] = jnp.ones((LANES,), jnp.int32)
        return _

    lax.fori_loop(0, NPW // LANES, compute_rows, None)

    hs = []
    fhs = []
    for j in range(NCHUNK):
        if j >= 2:
            hs[j - 2].wait()
        pltpu.sync_copy(feat_hbm.at[pl.ds(base + j * SCH, SCH)],
                        fbuf.at[j % 2, :, pl.ds(0, C)])
        h = pltpu.make_async_copy(
            fbuf.at[j % 2],
            canvas_hbm.at[rows2d.at[j]],
            sem,
        )
        h.start()
        hs.append(h)
        fh = pltpu.make_async_copy(
            ones2d.at[j],
            flags_hbm.at[rows2d.at[j]],
            fsem,
        )
        fh.start()
        fhs.append(fh)
    hs[-2].wait()
    hs[-1].wait()
    for fh in fhs:
        fh.wait()


def _scatter_sc(feat, b_arr, y_arr, x_arr, flags0):
    mesh = plsc.VectorSubcoreMesh(core_axis_name="c", subcore_axis_name="s")
    fn = _mpmd._mpmd_map(
        [(mesh, _scatter_body)],
        (jax.ShapeDtypeStruct((R, CW), jnp.float32),
         jax.ShapeDtypeStruct((R,), jnp.int32)),
        input_output_aliases={4: 1},
        scratch_types=[
            pltpu.VMEM((NPW,), jnp.int32),
            pltpu.VMEM((NPW,), jnp.int32),
            pltpu.VMEM((NPW,), jnp.int32),
            pltpu.VMEM((NCHUNK, SCH), jnp.int32),
            pltpu.VMEM((NCHUNK, SCH), jnp.int32),
            pltpu.VMEM((2, SCH, CW), jnp.float32),
            pltpu.SemaphoreType.DMA,
            pltpu.SemaphoreType.DMA,
        ],
        compiler_params=pltpu.CompilerParams(use_tc_tiling_on_sc=False),
        interpret=False,
        debug=False,
        cost_estimate=None,
        name="pillar_scatter_sc",
        metadata=None,
    )
    return fn(feat, b_arr, y_arr, x_arr, flags0)


def _retile_body(cv_ref, fl_ref, o_ref):
    f = fl_ref[0]
    xf = cv_ref[...].reshape(XG * NY, CW)
    for k in range(XG):
        sub = xf[k * NY:(k + 1) * NY, :C]
        xt = sub.T
        mk = (f[k] != 0)[None, :]
        o_ref[0, k] = jnp.where(mk, xt, jnp.zeros_like(xt))


def _retile(canvas3, flags2d):
    grid = B * NX // XG
    return pl.pallas_call(
        _retile_body,
        out_shape=jax.ShapeDtypeStruct((B, NX, C, NY), jnp.float32),
        grid=(grid,),
        in_specs=[
            pl.BlockSpec((XG * NY // 8, 8, 128), lambda i: (i, 0, 0)),
            pl.BlockSpec((1, XG, NY), lambda i: (i, 0, 0)),
        ],
        out_specs=pl.BlockSpec((1, XG, C, NY), lambda i: (i // (NX // XG), i % (NX // XG), 0, 0)),
    )(canvas3, flags2d)


def kernel(pillar_features, coors):
    coors = coors.astype(jnp.int32)
    pad = PPAD - P
    feat = jnp.concatenate(
        [pillar_features,
         jnp.broadcast_to(pillar_features[0], (pad, C))], axis=0)
    cpad = jnp.concatenate(
        [coors, jnp.broadcast_to(coors[0], (pad, 3))], axis=0)
    b_arr = cpad[:, 0]
    y_arr = cpad[:, 1]
    x_arr = cpad[:, 2]
    flags0 = _zero_flags()
    canvas, flags = _scatter_sc(feat, b_arr, y_arr, x_arr, flags0)
    canvas3 = canvas.reshape(R * CW // (8 * 128), 8, 128)
    out_t = _retile(canvas3, flags.reshape(B * NX // XG, XG, NY))
    return jnp.transpose(out_t, (0, 1, 3, 2))


# speedup vs baseline: 2.3252x; 1.0208x over previous
import jax
import jax.numpy as jnp
from jax import lax
from jax.experimental import pallas as pl
from jax.experimental.pallas import tpu as pltpu
from jax.experimental.pallas import tpu_sc as plsc
from jax._src.pallas import mpmd as _mpmd

NX, NY, C, B, P = 512, 512, 64, 4, 40000
R = B * NX * NY
CW = 128
NW = 32
PPAD = 40960
NPW = PPAD // NW
SCH = 128
NCHUNK = NPW // SCH
LANES = 16

XG = 4


def _flag_zero_body(o_ref):
    o_ref[...] = jnp.zeros_like(o_ref)


def _zero_flags():
    return pl.pallas_call(
        _flag_zero_body,
        out_shape=jax.ShapeDtypeStruct((R,), jnp.int32),
        grid=(16,),
        out_specs=pl.BlockSpec((R // 16,), lambda i: (i,)),
    )()


def _scatter_body(feat_hbm, b_hbm, y_hbm, x_hbm, flags0_hbm,
                  canvas_hbm, flags_hbm,
                  bv, yv, xv, rows2d, ones2d, fbuf, sem, fsem):
    del flags0_hbm
    wid = lax.axis_index("s") * 2 + lax.axis_index("c")
    base = wid * NPW
    pltpu.sync_copy(b_hbm.at[pl.ds(base, NPW)], bv)
    pltpu.sync_copy(y_hbm.at[pl.ds(base, NPW)], yv)
    pltpu.sync_copy(x_hbm.at[pl.ds(base, NPW)], xv)

    def compute_rows(i, _):
        off = i * LANES
        vb = bv[pl.ds(off, LANES)]
        vy = yv[pl.ds(off, LANES)]
        vx = xv[pl.ds(off, LANES)]
        row = vb * (NX * NY) + vx * NY + vy
        r = i // (SCH // LANES)
        c = (i % (SCH // LANES)) * LANES
        rows2d[r, pl.ds(c, LANES)] = row
        ones2d[r, pl.ds(c, LANES)] = jnp.ones((LANES,), jnp.int32)
        return _

    lax.fori_loop(0, NPW // LANES, compute_rows, None)

    hs = []
    fhs = []
    for j in range(NCHUNK):
        if j >= 2:
            hs[j - 2].wait()
        pltpu.sync_copy(feat_hbm.at[pl.ds(base + j * SCH, SCH)],
                        fbuf.at[j % 2])
        h = pltpu.make_async_copy(
            fbuf.at[j % 2],
            canvas_hbm.at[rows2d.at[j]],
            sem,
        )
        h.start()
        hs.append(h)
        fh = pltpu.make_async_copy(
            ones2d.at[j],
            flags_hbm.at[rows2d.at[j]],
            fsem,
        )
        fh.start()
        fhs.append(fh)
    hs[-2].wait()
    hs[-1].wait()
    for fh in fhs:
        fh.wait()


def _scatter_sc(feat, b_arr, y_arr, x_arr, flags0):
    mesh = plsc.VectorSubcoreMesh(core_axis_name="c", subcore_axis_name="s")
    fn = _mpmd._mpmd_map(
        [(mesh, _scatter_body)],
        (jax.ShapeDtypeStruct((R, CW), jnp.float32),
         jax.ShapeDtypeStruct((R,), jnp.int32)),
        input_output_aliases={4: 1},
        scratch_types=[
            pltpu.VMEM((NPW,), jnp.int32),
            pltpu.VMEM((NPW,), jnp.int32),
            pltpu.VMEM((NPW,), jnp.int32),
            pltpu.VMEM((NCHUNK, SCH), jnp.int32),
            pltpu.VMEM((NCHUNK, SCH), jnp.int32),
            pltpu.VMEM((2, SCH, CW), jnp.float32),
            pltpu.SemaphoreType.DMA,
            pltpu.SemaphoreType.DMA,
        ],
        compiler_params=pltpu.CompilerParams(use_tc_tiling_on_sc=False),
        interpret=False,
        debug=False,
        cost_estimate=None,
        name="pillar_scatter_sc",
        metadata=None,
    )
    return fn(feat, b_arr, y_arr, x_arr, flags0)


def _retile_body(cv_ref, fl_ref, o_ref):
    f = fl_ref[0]
    xf = cv_ref[...].reshape(XG * NY, CW)
    for k in range(XG):
        sub = xf[k * NY:(k + 1) * NY, :C]
        xt = sub.T
        mk = (f[k] != 0)[None, :]
        o_ref[0, k] = jnp.where(mk, xt, jnp.zeros_like(xt))


def _retile(canvas3, flags2d):
    grid = B * NX // XG
    return pl.pallas_call(
        _retile_body,
        out_shape=jax.ShapeDtypeStruct((B, NX, C, NY), jnp.float32),
        grid=(grid,),
        in_specs=[
            pl.BlockSpec((XG * NY // 8, 8, 128), lambda i: (i, 0, 0)),
            pl.BlockSpec((1, XG, NY), lambda i: (i, 0, 0)),
        ],
        out_specs=pl.BlockSpec((1, XG, C, NY), lambda i: (i // (NX // XG), i % (NX // XG), 0, 0)),
    )(canvas3, flags2d)


def kernel(pillar_features, coors):
    coors = coors.astype(jnp.int32)
    pad = PPAD - P
    feat = jnp.concatenate(
        [pillar_features,
         jnp.broadcast_to(pillar_features[0], (pad, C))], axis=0)
    feat = jnp.pad(feat, ((0, 0), (0, CW - C)))
    cpad = jnp.concatenate(
        [coors, jnp.broadcast_to(coors[0], (pad, 3))], axis=0)
    b_arr = cpad[:, 0]
    y_arr = cpad[:, 1]
    x_arr = cpad[:, 2]
    flags0 = _zero_flags()
    canvas, flags = _scatter_sc(feat, b_arr, y_arr, x_arr, flags0)
    canvas3 = canvas.reshape(R * CW // (8 * 128), 8, 128)
    out_t = _retile(canvas3, flags.reshape(B * NX // XG, XG, NY))
    return jnp.transpose(out_t, (0, 1, 3, 2))


# speedup vs baseline: 2.7433x; 1.1798x over previous
import jax
import jax.numpy as jnp
from jax import lax
from jax.experimental import pallas as pl
from jax.experimental.pallas import tpu as pltpu
from jax.experimental.pallas import tpu_sc as plsc
from jax._src.pallas import mpmd as _mpmd

NX, NY, C, B, P = 512, 512, 64, 4, 40000
R = B * NX * NY
CW = 128
NW = 32
PPAD = 40960
NPW = PPAD // NW
SCH = 128
NCHUNK = NPW // SCH
LANES = 16

XG = 4
ZBUF = 16384


def _flag_zero_body(o_ref):
    o_ref[...] = jnp.zeros_like(o_ref)


def _zero_flags():
    return pl.pallas_call(
        _flag_zero_body,
        out_shape=jax.ShapeDtypeStruct((R,), jnp.int32),
        grid=(16,),
        out_specs=pl.BlockSpec((R // 16,), lambda i: (i,)),
    )()


def _scatter_body(feat_hbm, b_hbm, y_hbm, x_hbm,
                  canvas_hbm, flags_hbm,
                  bv, yv, xv, rows2d, ones2d, fbuf, zbuf, shflags,
                  sem, fsem):
    cid = lax.axis_index("c")
    sid = lax.axis_index("s")
    wid = sid * 2 + cid
    base = wid * NPW
    pltpu.sync_copy(b_hbm.at[pl.ds(base, NPW)], bv)
    pltpu.sync_copy(y_hbm.at[pl.ds(base, NPW)], yv)
    pltpu.sync_copy(x_hbm.at[pl.ds(base, NPW)], xv)

    def zero_zbuf(i, _):
        zbuf[pl.ds(i * LANES, LANES)] = jnp.zeros((LANES,), jnp.int32)
        return _

    lax.fori_loop(0, ZBUF // LANES, zero_zbuf, None)

    def compute_rows(i, _):
        off = i * LANES
        vb = bv[pl.ds(off, LANES)]
        vy = yv[pl.ds(off, LANES)]
        vx = xv[pl.ds(off, LANES)]
        row = vb * (NX * NY) + vx * NY + vy
        r = i // (SCH // LANES)
        c = (i % (SCH // LANES)) * LANES
        rows2d[r, pl.ds(c, LANES)] = row
        ones2d[r, pl.ds(c, LANES)] = jnp.ones((LANES,), jnp.int32)
        return _

    lax.fori_loop(0, NPW // LANES, compute_rows, None)

    slab = R // 16
    for t in range(slab // ZBUF):
        pltpu.sync_copy(zbuf, shflags.at[pl.ds(sid * slab + t * ZBUF, ZBUF)])
    plsc.subcore_barrier()

    fhs = []
    for j in range(NCHUNK):
        fh = pltpu.make_async_copy(
            ones2d.at[j],
            shflags.at[rows2d.at[j]],
            fsem,
        )
        fh.start()
        fhs.append(fh)

    hs = []
    for j in range(NCHUNK):
        if j >= 2:
            hs[j - 2].wait()
        pltpu.sync_copy(feat_hbm.at[pl.ds(base + j * SCH, SCH)],
                        fbuf.at[j % 2])
        h = pltpu.make_async_copy(
            fbuf.at[j % 2],
            canvas_hbm.at[rows2d.at[j]],
            sem,
        )
        h.start()
        hs.append(h)
    hs[-2].wait()
    hs[-1].wait()
    for fh in fhs:
        fh.wait()
    plsc.subcore_barrier()
    pltpu.sync_copy(shflags.at[pl.ds(sid * slab, slab)],
                    flags_hbm.at[cid, pl.ds(sid * slab, slab)])


def _scatter_sc(feat, b_arr, y_arr, x_arr):
    mesh = plsc.VectorSubcoreMesh(core_axis_name="c", subcore_axis_name="s")
    fn = _mpmd._mpmd_map(
        [(mesh, _scatter_body)],
        (jax.ShapeDtypeStruct((R, CW), jnp.float32),
         jax.ShapeDtypeStruct((2, R), jnp.int32)),
        input_output_aliases={},
        scratch_types=[
            pltpu.VMEM((NPW,), jnp.int32),
            pltpu.VMEM((NPW,), jnp.int32),
            pltpu.VMEM((NPW,), jnp.int32),
            pltpu.VMEM((NCHUNK, SCH), jnp.int32),
            pltpu.VMEM((NCHUNK, SCH), jnp.int32),
            pltpu.VMEM((2, SCH, CW), jnp.float32),
            pltpu.VMEM((ZBUF,), jnp.int32),
            pltpu.VMEM_SHARED((R,), jnp.int32),
            pltpu.SemaphoreType.DMA,
            pltpu.SemaphoreType.DMA,
        ],
        compiler_params=pltpu.CompilerParams(use_tc_tiling_on_sc=False),
        interpret=False,
        debug=False,
        cost_estimate=None,
        name="pillar_scatter_sc",
        metadata=None,
    )
    return fn(feat, b_arr, y_arr, x_arr)


def _retile_body(cv_ref, fl_ref, o_ref):
    f = fl_ref[:, 0]
    xf = cv_ref[...].reshape(XG * NY, CW)
    for k in range(XG):
        sub = xf[k * NY:(k + 1) * NY, :C]
        xt = sub.T
        mk = ((f[0, k] | f[1, k]) != 0)[None, :]
        o_ref[0, k] = jnp.where(mk, xt, jnp.zeros_like(xt))


def _retile(canvas3, flags2d):
    grid = B * NX // XG
    return pl.pallas_call(
        _retile_body,
        out_shape=jax.ShapeDtypeStruct((B, NX, C, NY), jnp.float32),
        grid=(grid,),
        in_specs=[
            pl.BlockSpec((XG * NY // 8, 8, 128), lambda i: (i, 0, 0)),
            pl.BlockSpec((2, 1, XG, NY), lambda i: (0, i, 0, 0)),
        ],
        out_specs=pl.BlockSpec((1, XG, C, NY), lambda i: (i // (NX // XG), i % (NX // XG), 0, 0)),
    )(canvas3, flags2d)


def kernel(pillar_features, coors):
    coors = coors.astype(jnp.int32)
    pad = PPAD - P
    feat = jnp.concatenate(
        [pillar_features,
         jnp.broadcast_to(pillar_features[0], (pad, C))], axis=0)
    feat = jnp.pad(feat, ((0, 0), (0, CW - C)))
    cpad = jnp.concatenate(
        [coors, jnp.broadcast_to(coors[0], (pad, 3))], axis=0)
    b_arr = cpad[:, 0]
    y_arr = cpad[:, 1]
    x_arr = cpad[:, 2]
    canvas, flags = _scatter_sc(feat, b_arr, y_arr, x_arr)
    canvas3 = canvas.reshape(R * CW // (8 * 128), 8, 128)
    out_t = _retile(canvas3, flags.reshape(2, B * NX // XG, XG, NY))
    return jnp.transpose(out_t, (0, 1, 3, 2))


# speedup vs baseline: 3.3004x; 1.2031x over previous
import jax
import jax.numpy as jnp
from jax import lax
from jax.experimental import pallas as pl
from jax.experimental.pallas import tpu as pltpu
from jax.experimental.pallas import tpu_sc as plsc
from jax._src.pallas import mpmd as _mpmd

NX, NY, C, B, P = 512, 512, 64, 4, 40000
R = B * NX * NY
CW = 128
NW = 32
PPAD = 40960
NPW = PPAD // NW
SCH = 128
NCHUNK = NPW // SCH
LANES = 16

XG = 8
ZBUF = 16384


def _flag_zero_body(o_ref):
    o_ref[...] = jnp.zeros_like(o_ref)


def _zero_flags():
    return pl.pallas_call(
        _flag_zero_body,
        out_shape=jax.ShapeDtypeStruct((R,), jnp.int32),
        grid=(16,),
        out_specs=pl.BlockSpec((R // 16,), lambda i: (i,)),
    )()


def _scatter_body(feat_hbm, b_hbm, y_hbm, x_hbm,
                  canvas_hbm, flags_hbm,
                  bv, yv, xv, rows2d, ones2d, fbuf, zbuf, shflags,
                  sem, fsem):
    cid = lax.axis_index("c")
    sid = lax.axis_index("s")
    wid = sid * 2 + cid
    base = wid * NPW
    pltpu.sync_copy(b_hbm.at[pl.ds(base, NPW)], bv)
    pltpu.sync_copy(y_hbm.at[pl.ds(base, NPW)], yv)
    pltpu.sync_copy(x_hbm.at[pl.ds(base, NPW)], xv)

    def zero_zbuf(i, _):
        zbuf[pl.ds(i * LANES, LANES)] = jnp.zeros((LANES,), jnp.int32)
        return _

    lax.fori_loop(0, ZBUF // LANES, zero_zbuf, None)

    def compute_rows(i, _):
        off = i * LANES
        vb = bv[pl.ds(off, LANES)]
        vy = yv[pl.ds(off, LANES)]
        vx = xv[pl.ds(off, LANES)]
        row = vb * (NX * NY) + vx * NY + vy
        r = i // (SCH // LANES)
        c = (i % (SCH // LANES)) * LANES
        rows2d[r, pl.ds(c, LANES)] = row
        ones2d[r, pl.ds(c, LANES)] = jnp.ones((LANES,), jnp.int32)
        return _

    lax.fori_loop(0, NPW // LANES, compute_rows, None)

    slab = R // 16
    for t in range(slab // ZBUF):
        pltpu.sync_copy(zbuf, shflags.at[pl.ds(sid * slab + t * ZBUF, ZBUF)])
    plsc.subcore_barrier()

    fhs = []
    for j in range(NCHUNK):
        fh = pltpu.make_async_copy(
            ones2d.at[j],
            shflags.at[rows2d.at[j]],
            fsem,
        )
        fh.start()
        fhs.append(fh)

    hs = []
    for j in range(NCHUNK):
        if j >= 2:
            hs[j - 2].wait()
        pltpu.sync_copy(feat_hbm.at[pl.ds(base + j * SCH, SCH)],
                        fbuf.at[j % 2])
        h = pltpu.make_async_copy(
            fbuf.at[j % 2],
            canvas_hbm.at[rows2d.at[j]],
            sem,
        )
        h.start()
        hs.append(h)
    hs[-2].wait()
    hs[-1].wait()
    for fh in fhs:
        fh.wait()
    plsc.subcore_barrier()
    pltpu.sync_copy(shflags.at[pl.ds(sid * slab, slab)],
                    flags_hbm.at[cid, pl.ds(sid * slab, slab)])


def _scatter_sc(feat, b_arr, y_arr, x_arr):
    mesh = plsc.VectorSubcoreMesh(core_axis_name="c", subcore_axis_name="s")
    fn = _mpmd._mpmd_map(
        [(mesh, _scatter_body)],
        (jax.ShapeDtypeStruct((R, CW), jnp.float32),
         jax.ShapeDtypeStruct((2, R), jnp.int32)),
        input_output_aliases={},
        scratch_types=[
            pltpu.VMEM((NPW,), jnp.int32),
            pltpu.VMEM((NPW,), jnp.int32),
            pltpu.VMEM((NPW,), jnp.int32),
            pltpu.VMEM((NCHUNK, SCH), jnp.int32),
            pltpu.VMEM((NCHUNK, SCH), jnp.int32),
            pltpu.VMEM((2, SCH, CW), jnp.float32),
            pltpu.VMEM((ZBUF,), jnp.int32),
            pltpu.VMEM_SHARED((R,), jnp.int32),
            pltpu.SemaphoreType.DMA,
            pltpu.SemaphoreType.DMA,
        ],
        compiler_params=pltpu.CompilerParams(use_tc_tiling_on_sc=False),
        interpret=False,
        debug=False,
        cost_estimate=None,
        name="pillar_scatter_sc",
        metadata=None,
    )
    return fn(feat, b_arr, y_arr, x_arr)


def _retile_body(cv_ref, fl_ref, o_ref):
    f = fl_ref[:, 0]
    xf = cv_ref[...].reshape(XG * NY, CW)
    for k in range(XG):
        sub = xf[k * NY:(k + 1) * NY, :C]
        xt = sub.T
        mk = ((f[0, k] | f[1, k]) != 0)[None, :]
        o_ref[0, k] = jnp.where(mk, xt, jnp.zeros_like(xt))


def _retile(canvas3, flags2d):
    grid = B * NX // XG
    return pl.pallas_call(
        _retile_body,
        out_shape=jax.ShapeDtypeStruct((B, NX, C, NY), jnp.float32),
        grid=(grid,),
        in_specs=[
            pl.BlockSpec((XG * NY // 8, 8, 128), lambda i: (i, 0, 0)),
            pl.BlockSpec((2, 1, XG, NY), lambda i: (0, i, 0, 0)),
        ],
        out_specs=pl.BlockSpec((1, XG, C, NY), lambda i: (i // (NX // XG), i % (NX // XG), 0, 0)),
    )(canvas3, flags2d)


def kernel(pillar_features, coors):
    coors = coors.astype(jnp.int32)
    pad = PPAD - P
    feat = jnp.concatenate(
        [pillar_features,
         jnp.broadcast_to(pillar_features[0], (pad, C))], axis=0)
    feat = jnp.pad(feat, ((0, 0), (0, CW - C)))
    cpad = jnp.concatenate(
        [coors, jnp.broadcast_to(coors[0], (pad, 3))], axis=0)
    b_arr = cpad[:, 0]
    y_arr = cpad[:, 1]
    x_arr = cpad[:, 2]
    canvas, flags = _scatter_sc(feat, b_arr, y_arr, x_arr)
    canvas3 = canvas.reshape(R * CW // (8 * 128), 8, 128)
    out_t = _retile(canvas3, flags.reshape(2, B * NX // XG, XG, NY))
    return jnp.transpose(out_t, (0, 1, 3, 2))
